# baseline (speedup 1.0000x reference)
import functools

import jax
import jax.numpy as jnp
from jax import lax
from jax.experimental import pallas as pl
from jax.experimental.pallas import tpu as pltpu
from jax.experimental.pallas import tpu_sc as plsc

_D = 256
_H = 8
_L = 4
_P = 4
_HD = 32
_NB = 2
_LQ = 5440
_TQ = _NB * _LQ
_NW = 32
_RPW = _TQ // _NW

_PREP_BLK = 320
_PREP_GRID = _TQ // _PREP_BLK
_BLKS_PER_B = _LQ // _PREP_BLK

_OUT_BLK = 640
_OUT_GRID = _TQ // _OUT_BLK


def _prep_body(q_ref, rp_ref, f_ref, w_ref, b_ref, idx_ref, wgt_ref, val_ref):
    batch = pl.program_id(0) // _BLKS_PER_B

    x = f_ref[...]
    ci = lax.broadcasted_iota(jnp.int32, (256, 128), 0)
    cj = lax.broadcasted_iota(jnp.int32, (256, 128), 1)
    locol = (cj // 16) * 32 + (cj % 16)
    pmat_lo = jnp.where(ci == locol, 1.0, 0.0).astype(jnp.float32)
    pmat_hi = jnp.where(ci == locol + 16, 1.0, 0.0).astype(jnp.float32)
    lo = jnp.dot(x, pmat_lo, preferred_element_type=jnp.float32)
    hi = jnp.dot(x, pmat_hi, preferred_element_type=jnp.float32)
    lo16 = lax.bitcast_convert_type(lo.astype(jnp.bfloat16), jnp.uint16)
    hi16 = lax.bitcast_convert_type(hi.astype(jnp.bfloat16), jnp.uint16)
    val_ref[...] = lo16.astype(jnp.uint32) | (hi16.astype(jnp.uint32) << 16)

    proj = jnp.dot(q_ref[...], w_ref[...].T, preferred_element_type=jnp.float32)
    proj = proj + b_ref[...]
    offx = proj[:, 0:128]
    offy = proj[:, 128:256]
    logits = proj[:, 256:384]

    logits = logits - jnp.max(logits, axis=1, keepdims=True)
    e = jnp.exp(logits)
    ii = lax.broadcasted_iota(jnp.int32, (128, 128), 0)
    jj = lax.broadcasted_iota(jnp.int32, (128, 128), 1)
    grp = jnp.where((ii // 16) == (jj // 16), 1.0, 0.0).astype(jnp.float32)
    gsum = jnp.dot(e, grp, preferred_element_type=jnp.float32)
    attn = e / gsum

    col = lax.broadcasted_iota(jnp.int32, (1, 128), 1)
    lvl = (col >> 2) & 3
    head = col >> 4
    wl = 64 >> lvl
    hl = 64 >> lvl
    start = jnp.where(lvl == 0, 0,
            jnp.where(lvl == 1, 4096,
            jnp.where(lvl == 2, 5120, 5376)))
    wl_f = wl.astype(jnp.float32)
    hl_f = hl.astype(jnp.float32)

    ix = rp_ref[:, 0:1] * wl_f + offx - 0.5
    iy = rp_ref[:, 1:2] * hl_f + offy - 0.5
    x0f = jnp.floor(ix)
    y0f = jnp.floor(iy)
    wx1 = ix - x0f
    wx0 = 1.0 - wx1
    wy1 = iy - y0f
    wy0 = 1.0 - wy1
    x0 = x0f.astype(jnp.int32)
    y0 = y0f.astype(jnp.int32)

    one = jnp.float32(1.0)
    zero = jnp.float32(0.0)
    xp = jnp.clip(x0, 0, wl - 2)
    x1 = x0 + 1
    s0 = (wx0 * jnp.where(x0 == xp, one, zero)
          + wx1 * jnp.where(x1 == xp, one, zero))
    s1 = (wx0 * jnp.where(x0 == xp + 1, one, zero)
          + wx1 * jnp.where(x1 == xp + 1, one, zero))
    for yp, wy in ((0, wy0), (1, wy1)):
        yv = y0 + yp
        yvalid = jnp.where((yv >= 0) & (yv < hl), one, zero)
        yc = jnp.clip(yv, 0, hl - 1)
        spatial = start + yc * wl + xp
        row = (batch * _LQ + spatial) * _H + head
        idx_ref[:, yp * 128:(yp + 1) * 128] = row
        wgt_ref[:, (yp * 2) * 128:(yp * 2 + 1) * 128] = attn * s0 * wy * yvalid
        wgt_ref[:, (yp * 2 + 1) * 128:(yp * 2 + 2) * 128] = (
            attn * s1 * wy * yvalid)


def _prep(q2d, rp2d, f2d, wcat, bcat):
    return pl.pallas_call(
        _prep_body,
        grid=(_PREP_GRID,),
        in_specs=[
            pl.BlockSpec((_PREP_BLK, _D), lambda i: (i, 0)),
            pl.BlockSpec((_PREP_BLK, 2), lambda i: (i, 0)),
            pl.BlockSpec((_PREP_BLK, _D), lambda i: (i, 0)),
            pl.BlockSpec((384, _D), lambda i: (0, 0)),
            pl.BlockSpec((1, 384), lambda i: (0, 0)),
        ],
        out_specs=[
            pl.BlockSpec((_PREP_BLK, 256), lambda i: (i, 0)),
            pl.BlockSpec((_PREP_BLK, 512), lambda i: (i, 0)),
            pl.BlockSpec((_PREP_BLK, 128), lambda i: (i, 0)),
        ],
        out_shape=[
            jax.ShapeDtypeStruct((_TQ, 256), jnp.int32),
            jax.ShapeDtypeStruct((_TQ, 512), jnp.float32),
            jax.ShapeDtypeStruct((_TQ, 128), jnp.uint32),
        ],
    )(q2d, rp2d, f2d, wcat, bcat)


def _sc_body(val_hbm, idx_hbm, w_hbm, out_hbm, idx_v, w_v, g_v, out_v,
             isem, gsem, osem):
    wid = lax.axis_index("s") * 2 + lax.axis_index("c")
    r0 = wid * _RPW

    def fire_gathers(r, islot, gslot):
        for yp in range(2):
            pltpu.async_copy(
                val_hbm.at[idx_v.at[islot, pl.ds(yp * 128, 128)]],
                g_v.at[gslot, pl.ds(yp * 128, 128)],
                gsem,
            )

    def drain_gathers(islot, gslot):
        for yp in range(2):
            pltpu.make_async_copy(
                val_hbm.at[idx_v.at[islot, pl.ds(yp * 128, 128)]],
                g_v.at[gslot, pl.ds(yp * 128, 128)],
                gsem,
            ).wait()

    def compute(r, islot, gslot, oslot):
        for h in range(8):
            accs = []
            for c in range(4):
                yp, sl = c // 2, c % 2
                wv = w_v[islot, pl.ds(c * 128 + h * 16, 16)]
                a0 = jnp.zeros((16,), jnp.float32)
                a1 = jnp.zeros((16,), jnp.float32)
                for k in range(16):
                    wj = wv[k]
                    g32 = g_v[gslot, yp * 128 + h * 16 + k,
                              pl.ds(sl * 16, 16)]
                    ge = lax.bitcast_convert_type(g32 << 16, jnp.float32)
                    go = lax.bitcast_convert_type(
                        g32 & jnp.uint32(0xFFFF0000), jnp.float32)
                    a0 = a0 + wj * ge
                    a1 = a1 + wj * go
                accs.append((a0, a1))
            out_v[oslot, h, pl.ds(0, 16)] = (
                (accs[0][0] + accs[1][0]) + (accs[2][0] + accs[3][0]))
            out_v[oslot, h, pl.ds(16, 16)] = (
                (accs[0][1] + accs[1][1]) + (accs[2][1] + accs[3][1]))
        pltpu.async_copy(out_v.at[oslot], out_hbm.at[r], osem)

    for p in range(5):
        pltpu.sync_copy(idx_hbm.at[r0 + p], idx_v.at[p])
        pltpu.sync_copy(w_hbm.at[r0 + p], w_v.at[p])
    for p in range(3):
        fire_gathers(r0 + p, p, p)

    def step(i, carry):
        r = r0 + i
        f = i + 3
        islot = lax.rem(i, 8)
        fslot = lax.rem(f, 8)
        pslot = lax.rem(i + 5, 8)
        gslot = lax.rem(i, 4)
        fgslot = lax.rem(f, 4)
        oslot = lax.rem(i, 2)

        drain_gathers(islot, gslot)

        @pl.when(i + 5 < _RPW)
        def _():
            pltpu.async_copy(idx_hbm.at[r + 5], idx_v.at[pslot], isem)
            pltpu.async_copy(w_hbm.at[r + 5], w_v.at[pslot], isem)

        @pl.when((f < _RPW) & (f >= 5))
        def _():
            pltpu.make_async_copy(
                idx_hbm.at[r + 3], idx_v.at[fslot], isem).wait()
            pltpu.make_async_copy(
                w_hbm.at[r + 3], w_v.at[fslot], isem).wait()

        @pl.when(f < _RPW)
        def _():
            fire_gathers(r + 3, fslot, fgslot)

        @pl.when(i >= 2)
        def _():
            pltpu.make_async_copy(
                out_v.at[oslot], out_hbm.at[r - 2], osem).wait()

        compute(r, islot, gslot, oslot)
        return carry

    lax.fori_loop(0, _RPW, step, 0)

    pltpu.make_async_copy(
        out_v.at[0], out_hbm.at[r0 + _RPW - 2], osem).wait()
    pltpu.make_async_copy(
        out_v.at[1], out_hbm.at[r0 + _RPW - 1], osem).wait()


@functools.cache
def _sc_gather_fn():
    return pl.kernel(
        _sc_body,
        out_type=jax.ShapeDtypeStruct((_TQ, _H, _HD), jnp.float32),
        mesh=plsc.VectorSubcoreMesh(core_axis_name="c", subcore_axis_name="s"),
        scratch_types=[
            pltpu.VMEM((8, 256), jnp.int32),
            pltpu.VMEM((8, 512), jnp.float32),
            pltpu.VMEM((4, 256, _HD), jnp.uint32),
            pltpu.VMEM((2, _H, _HD), jnp.float32),
            pltpu.SemaphoreType.DMA,
            pltpu.SemaphoreType.DMA,
            pltpu.SemaphoreType.DMA,
        ],
        compiler_params=pltpu.CompilerParams(use_tc_tiling_on_sc=False),
    )


def _sc_gather(val, idx, wgt):
    return _sc_gather_fn()(val, idx, wgt)


def _outproj_body(x_ref, w_ref, b_ref, o_ref):
    o_ref[...] = jnp.dot(x_ref[...], w_ref[...].T,
                         preferred_element_type=jnp.float32) + b_ref[...]


def _outproj(x2d, w_out, b_out2d):
    return pl.pallas_call(
        _outproj_body,
        grid=(_OUT_GRID,),
        in_specs=[
            pl.BlockSpec((_OUT_BLK, _D), lambda i: (i, 0)),
            pl.BlockSpec((_D, _D), lambda i: (0, 0)),
            pl.BlockSpec((1, _D), lambda i: (0, 0)),
        ],
        out_specs=pl.BlockSpec((_OUT_BLK, _D), lambda i: (i, 0)),
        out_shape=jax.ShapeDtypeStruct((_TQ, _D), jnp.float32),
    )(x2d, w_out, b_out2d)


def kernel(query, reference_points, input_flatten, spatial_shapes,
           level_start_index, W_off, b_off, W_att, b_att, W_out, b_out):
    q2d = query.reshape(_TQ, _D)
    rp2d = reference_points.reshape(_TQ, 2).astype(jnp.float32)

    wcat = jnp.concatenate([W_off[0::2], W_off[1::2], W_att], axis=0)
    bcat = jnp.concatenate([b_off[0::2], b_off[1::2], b_att])[None, :]

    f2d = input_flatten.reshape(_TQ, _D)
    idx, wgt, val2d = _prep(q2d, rp2d, f2d, wcat, bcat)

    v3 = val2d.reshape(_TQ, _H, _HD // 2)
    vnext = jnp.concatenate([v3[1:], v3[:1]], axis=0)
    val = jnp.concatenate([v3, vnext], axis=-1).reshape(_TQ * _H, _HD)
    sampled = _sc_gather(val, idx, wgt)

    out = _outproj(sampled.reshape(_TQ, _D), W_out, b_out[None, :])
    return out.reshape(_NB, _LQ, _D)


# speedup vs baseline: 1.1889x; 1.1889x over previous
import functools

import jax
import jax.numpy as jnp
from jax import lax
from jax.experimental import pallas as pl
from jax.experimental.pallas import tpu as pltpu
from jax.experimental.pallas import tpu_sc as plsc

_D = 256
_H = 8
_L = 4
_P = 4
_HD = 32
_NB = 2
_LQ = 5440
_TQ = _NB * _LQ
_NW = 32
_RPW = _TQ // _NW

_PREP_BLK = 320
_PREP_GRID = _TQ // _PREP_BLK
_BLKS_PER_B = _LQ // _PREP_BLK

_OUT_BLK = 640
_OUT_GRID = _TQ // _OUT_BLK


def _prep_body(q_ref, rp_ref, f_ref, fn_ref, w_ref, b_ref,
               idx_ref, wgt_ref, val_ref):
    batch = pl.program_id(0) // _BLKS_PER_B

    x = f_ref[...]
    xshift = jnp.concatenate([x[1:], fn_ref[pl.ds(0, 1), :]], axis=0)
    xcat = jnp.concatenate([x, xshift], axis=1)
    ci = lax.broadcasted_iota(jnp.int32, (512, 256), 0)
    cj = lax.broadcasted_iota(jnp.int32, (512, 256), 1)
    locol = ((cj % 32) // 16) * 256 + (cj // 32) * 32 + (cj % 16)
    pmat_lo = jnp.where(ci == locol, 1.0, 0.0).astype(jnp.float32)
    pmat_hi = jnp.where(ci == locol + 16, 1.0, 0.0).astype(jnp.float32)
    lo = jnp.dot(xcat, pmat_lo, preferred_element_type=jnp.float32)
    hi = jnp.dot(xcat, pmat_hi, preferred_element_type=jnp.float32)
    lo16 = lax.bitcast_convert_type(lo.astype(jnp.bfloat16), jnp.uint16)
    hi16 = lax.bitcast_convert_type(hi.astype(jnp.bfloat16), jnp.uint16)
    val_ref[...] = lo16.astype(jnp.uint32) | (hi16.astype(jnp.uint32) << 16)

    proj = jnp.dot(q_ref[...], w_ref[...].T, preferred_element_type=jnp.float32)
    proj = proj + b_ref[...]
    offx = proj[:, 0:128]
    offy = proj[:, 128:256]
    logits = proj[:, 256:384]

    logits = logits - jnp.max(logits, axis=1, keepdims=True)
    e = jnp.exp(logits)
    ii = lax.broadcasted_iota(jnp.int32, (128, 128), 0)
    jj = lax.broadcasted_iota(jnp.int32, (128, 128), 1)
    grp = jnp.where((ii // 16) == (jj // 16), 1.0, 0.0).astype(jnp.float32)
    gsum = jnp.dot(e, grp, preferred_element_type=jnp.float32)
    attn = e / gsum

    col = lax.broadcasted_iota(jnp.int32, (1, 128), 1)
    lvl = (col >> 2) & 3
    head = col >> 4
    wl = 64 >> lvl
    hl = 64 >> lvl
    start = jnp.where(lvl == 0, 0,
            jnp.where(lvl == 1, 4096,
            jnp.where(lvl == 2, 5120, 5376)))
    wl_f = wl.astype(jnp.float32)
    hl_f = hl.astype(jnp.float32)

    ix = rp_ref[:, 0:1] * wl_f + offx - 0.5
    iy = rp_ref[:, 1:2] * hl_f + offy - 0.5
    x0f = jnp.floor(ix)
    y0f = jnp.floor(iy)
    wx1 = ix - x0f
    wx0 = 1.0 - wx1
    wy1 = iy - y0f
    wy0 = 1.0 - wy1
    x0 = x0f.astype(jnp.int32)
    y0 = y0f.astype(jnp.int32)

    one = jnp.float32(1.0)
    zero = jnp.float32(0.0)
    xp = jnp.clip(x0, 0, wl - 2)
    x1 = x0 + 1
    s0 = (wx0 * jnp.where(x0 == xp, one, zero)
          + wx1 * jnp.where(x1 == xp, one, zero))
    s1 = (wx0 * jnp.where(x0 == xp + 1, one, zero)
          + wx1 * jnp.where(x1 == xp + 1, one, zero))
    for yp, wy in ((0, wy0), (1, wy1)):
        yv = y0 + yp
        yvalid = jnp.where((yv >= 0) & (yv < hl), one, zero)
        yc = jnp.clip(yv, 0, hl - 1)
        spatial = start + yc * wl + xp
        row = (batch * _LQ + spatial) * _H + head
        idx_ref[:, yp * 128:(yp + 1) * 128] = row
        wgt_ref[:, (yp * 2) * 128:(yp * 2 + 1) * 128] = attn * s0 * wy * yvalid
        wgt_ref[:, (yp * 2 + 1) * 128:(yp * 2 + 2) * 128] = (
            attn * s1 * wy * yvalid)


def _prep(q2d, rp2d, f2d, wcat, bcat):
    return pl.pallas_call(
        _prep_body,
        grid=(_PREP_GRID,),
        in_specs=[
            pl.BlockSpec((_PREP_BLK, _D), lambda i: (i, 0)),
            pl.BlockSpec((_PREP_BLK, 2), lambda i: (i, 0)),
            pl.BlockSpec((_PREP_BLK, _D), lambda i: (i, 0)),
            pl.BlockSpec((_PREP_BLK, _D), lambda i: (i + 1, 0)),
            pl.BlockSpec((384, _D), lambda i: (0, 0)),
            pl.BlockSpec((1, 384), lambda i: (0, 0)),
        ],
        out_specs=[
            pl.BlockSpec((_PREP_BLK, 256), lambda i: (i, 0)),
            pl.BlockSpec((_PREP_BLK, 512), lambda i: (i, 0)),
            pl.BlockSpec((_PREP_BLK, 256), lambda i: (i, 0)),
        ],
        out_shape=[
            jax.ShapeDtypeStruct((_TQ, 256), jnp.int32),
            jax.ShapeDtypeStruct((_TQ, 512), jnp.float32),
            jax.ShapeDtypeStruct((_TQ, 256), jnp.uint32),
        ],
    )(q2d, rp2d, f2d, f2d, wcat, bcat)


def _sc_body(val_hbm, idx_hbm, w_hbm, out_hbm, idx_v, w_v, g_v, out_v,
             isem, gsem, osem):
    wid = lax.axis_index("s") * 2 + lax.axis_index("c")
    r0 = wid * _RPW

    def fire_gathers(r, islot, gslot):
        for yp in range(2):
            pltpu.async_copy(
                val_hbm.at[idx_v.at[islot, pl.ds(yp * 128, 128)]],
                g_v.at[gslot, pl.ds(yp * 128, 128)],
                gsem,
            )

    def drain_gathers(islot, gslot):
        for yp in range(2):
            pltpu.make_async_copy(
                val_hbm.at[idx_v.at[islot, pl.ds(yp * 128, 128)]],
                g_v.at[gslot, pl.ds(yp * 128, 128)],
                gsem,
            ).wait()

    def compute(r, islot, gslot, oslot):
        for h in range(8):
            accs = []
            for c in range(4):
                yp, sl = c // 2, c % 2
                wv = w_v[islot, pl.ds(c * 128 + h * 16, 16)]
                a0 = jnp.zeros((16,), jnp.float32)
                a1 = jnp.zeros((16,), jnp.float32)
                for k in range(16):
                    wj = wv[k]
                    g32 = g_v[gslot, yp * 128 + h * 16 + k,
                              pl.ds(sl * 16, 16)]
                    ge = lax.bitcast_convert_type(g32 << 16, jnp.float32)
                    go = lax.bitcast_convert_type(
                        g32 & jnp.uint32(0xFFFF0000), jnp.float32)
                    a0 = a0 + wj * ge
                    a1 = a1 + wj * go
                accs.append((a0, a1))
            out_v[oslot, h, pl.ds(0, 16)] = (
                (accs[0][0] + accs[1][0]) + (accs[2][0] + accs[3][0]))
            out_v[oslot, h, pl.ds(16, 16)] = (
                (accs[0][1] + accs[1][1]) + (accs[2][1] + accs[3][1]))
        pltpu.async_copy(out_v.at[oslot], out_hbm.at[r], osem)

    for p in range(5):
        pltpu.sync_copy(idx_hbm.at[r0 + p], idx_v.at[p])
        pltpu.sync_copy(w_hbm.at[r0 + p], w_v.at[p])
    for p in range(3):
        fire_gathers(r0 + p, p, p)

    def step(i, carry):
        r = r0 + i
        f = i + 3
        islot = lax.rem(i, 8)
        fslot = lax.rem(f, 8)
        pslot = lax.rem(i + 5, 8)
        gslot = lax.rem(i, 4)
        fgslot = lax.rem(f, 4)
        oslot = lax.rem(i, 2)

        drain_gathers(islot, gslot)

        @pl.when(i + 5 < _RPW)
        def _():
            pltpu.async_copy(idx_hbm.at[r + 5], idx_v.at[pslot], isem)
            pltpu.async_copy(w_hbm.at[r + 5], w_v.at[pslot], isem)

        @pl.when((f < _RPW) & (f >= 5))
        def _():
            pltpu.make_async_copy(
                idx_hbm.at[r + 3], idx_v.at[fslot], isem).wait()
            pltpu.make_async_copy(
                w_hbm.at[r + 3], w_v.at[fslot], isem).wait()

        @pl.when(f < _RPW)
        def _():
            fire_gathers(r + 3, fslot, fgslot)

        @pl.when(i >= 2)
        def _():
            pltpu.make_async_copy(
                out_v.at[oslot], out_hbm.at[r - 2], osem).wait()

        compute(r, islot, gslot, oslot)
        return carry

    lax.fori_loop(0, _RPW, step, 0)

    pltpu.make_async_copy(
        out_v.at[0], out_hbm.at[r0 + _RPW - 2], osem).wait()
    pltpu.make_async_copy(
        out_v.at[1], out_hbm.at[r0 + _RPW - 1], osem).wait()


@functools.cache
def _sc_gather_fn():
    return pl.kernel(
        _sc_body,
        out_type=jax.ShapeDtypeStruct((_TQ, _H, _HD), jnp.float32),
        mesh=plsc.VectorSubcoreMesh(core_axis_name="c", subcore_axis_name="s"),
        scratch_types=[
            pltpu.VMEM((8, 256), jnp.int32),
            pltpu.VMEM((8, 512), jnp.float32),
            pltpu.VMEM((4, 256, _HD), jnp.uint32),
            pltpu.VMEM((2, _H, _HD), jnp.float32),
            pltpu.SemaphoreType.DMA,
            pltpu.SemaphoreType.DMA,
            pltpu.SemaphoreType.DMA,
        ],
        compiler_params=pltpu.CompilerParams(use_tc_tiling_on_sc=False),
    )


def _sc_gather(val, idx, wgt):
    return _sc_gather_fn()(val, idx, wgt)


def _outproj_body(x_ref, w_ref, b_ref, o_ref):
    o_ref[...] = jnp.dot(x_ref[...], w_ref[...].T,
                         preferred_element_type=jnp.float32) + b_ref[...]


def _outproj(x2d, w_out, b_out2d):
    return pl.pallas_call(
        _outproj_body,
        grid=(_OUT_GRID,),
        in_specs=[
            pl.BlockSpec((_OUT_BLK, _D), lambda i: (i, 0)),
            pl.BlockSpec((_D, _D), lambda i: (0, 0)),
            pl.BlockSpec((1, _D), lambda i: (0, 0)),
        ],
        out_specs=pl.BlockSpec((_OUT_BLK, _D), lambda i: (i, 0)),
        out_shape=jax.ShapeDtypeStruct((_TQ, _D), jnp.float32),
    )(x2d, w_out, b_out2d)


def kernel(query, reference_points, input_flatten, spatial_shapes,
           level_start_index, W_off, b_off, W_att, b_att, W_out, b_out):
    q2d = query.reshape(_TQ, _D)
    rp2d = reference_points.reshape(_TQ, 2).astype(jnp.float32)

    wcat = jnp.concatenate([W_off[0::2], W_off[1::2], W_att], axis=0)
    bcat = jnp.concatenate([b_off[0::2], b_off[1::2], b_att])[None, :]

    f2d = input_flatten.reshape(_TQ, _D)
    idx, wgt, val2d = _prep(q2d, rp2d, f2d, wcat, bcat)

    val = val2d.reshape(_TQ * _H, _HD)
    sampled = _sc_gather(val, idx, wgt)

    out = _outproj(sampled.reshape(_TQ, _D), W_out, b_out[None, :])
    return out.reshape(_NB, _LQ, _D)


# speedup vs baseline: 1.2144x; 1.0214x over previous
import functools

import jax
import jax.numpy as jnp
from jax import lax
from jax.experimental import pallas as pl
from jax.experimental.pallas import tpu as pltpu
from jax.experimental.pallas import tpu_sc as plsc

_D = 256
_H = 8
_L = 4
_P = 4
_HD = 32
_NB = 2
_LQ = 5440
_TQ = _NB * _LQ
_NW = 32
_RPW = _TQ // _NW

_PREP_BLK = 320
_PREP_GRID = _TQ // _PREP_BLK
_BLKS_PER_B = _LQ // _PREP_BLK

_OUT_BLK = 640
_OUT_GRID = _TQ // _OUT_BLK


def _prep_body(q_ref, rp_ref, f_ref, fn_ref, w_ref, b_ref,
               idx_ref, wgt_ref, val_ref):
    batch = pl.program_id(0) // _BLKS_PER_B

    x = f_ref[...]
    xshift = jnp.concatenate([x[1:], fn_ref[pl.ds(0, 1), :]], axis=0)
    xcat = jnp.concatenate([x, xshift], axis=1)
    ci = lax.broadcasted_iota(jnp.int32, (512, 256), 0)
    cj = lax.broadcasted_iota(jnp.int32, (512, 256), 1)
    locol = ((cj % 32) // 16) * 256 + (cj // 32) * 32 + (cj % 16)
    pmat_lo = jnp.where(ci == locol, 1.0, 0.0).astype(jnp.float32)
    pmat_hi = jnp.where(ci == locol + 16, 1.0, 0.0).astype(jnp.float32)
    lo = jnp.dot(xcat, pmat_lo, preferred_element_type=jnp.float32)
    hi = jnp.dot(xcat, pmat_hi, preferred_element_type=jnp.float32)
    lo16 = lax.bitcast_convert_type(lo.astype(jnp.bfloat16), jnp.uint16)
    hi16 = lax.bitcast_convert_type(hi.astype(jnp.bfloat16), jnp.uint16)
    val_ref[...] = lo16.astype(jnp.uint32) | (hi16.astype(jnp.uint32) << 16)

    proj = jnp.dot(q_ref[...], w_ref[...].T, preferred_element_type=jnp.float32)
    proj = proj + b_ref[...]
    offx = proj[:, 0:128]
    offy = proj[:, 128:256]
    logits = proj[:, 256:384]

    logits = logits - jnp.max(logits, axis=1, keepdims=True)
    e = jnp.exp(logits)
    ii = lax.broadcasted_iota(jnp.int32, (128, 128), 0)
    jj = lax.broadcasted_iota(jnp.int32, (128, 128), 1)
    grp = jnp.where((ii // 16) == (jj // 16), 1.0, 0.0).astype(jnp.float32)
    gsum = jnp.dot(e, grp, preferred_element_type=jnp.float32)
    attn = e / gsum

    col = lax.broadcasted_iota(jnp.int32, (1, 128), 1)
    lvl = (col >> 2) & 3
    head = col >> 4
    wl = 64 >> lvl
    hl = 64 >> lvl
    start = jnp.where(lvl == 0, 0,
            jnp.where(lvl == 1, 4096,
            jnp.where(lvl == 2, 5120, 5376)))
    wl_f = wl.astype(jnp.float32)
    hl_f = hl.astype(jnp.float32)

    ix = rp_ref[:, 0:1] * wl_f + offx - 0.5
    iy = rp_ref[:, 1:2] * hl_f + offy - 0.5
    x0f = jnp.floor(ix)
    y0f = jnp.floor(iy)
    wx1 = ix - x0f
    wx0 = 1.0 - wx1
    wy1 = iy - y0f
    wy0 = 1.0 - wy1
    x0 = x0f.astype(jnp.int32)
    y0 = y0f.astype(jnp.int32)

    one = jnp.float32(1.0)
    zero = jnp.float32(0.0)
    xp = jnp.clip(x0, 0, wl - 2)
    x1 = x0 + 1
    s0 = (wx0 * jnp.where(x0 == xp, one, zero)
          + wx1 * jnp.where(x1 == xp, one, zero))
    s1 = (wx0 * jnp.where(x0 == xp + 1, one, zero)
          + wx1 * jnp.where(x1 == xp + 1, one, zero))
    for yp, wy in ((0, wy0), (1, wy1)):
        yv = y0 + yp
        yvalid = jnp.where((yv >= 0) & (yv < hl), one, zero)
        yc = jnp.clip(yv, 0, hl - 1)
        spatial = start + yc * wl + xp
        row = (batch * _LQ + spatial) * _H + head
        idx_ref[:, yp] = row.reshape(_PREP_BLK // 8, 8, 128)
        wgt_ref[:, yp * 2] = (attn * s0 * wy * yvalid).reshape(
            _PREP_BLK // 8, 8, 128)
        wgt_ref[:, yp * 2 + 1] = (attn * s1 * wy * yvalid).reshape(
            _PREP_BLK // 8, 8, 128)


def _prep(q2d, rp2d, f2d, wcat, bcat):
    return pl.pallas_call(
        _prep_body,
        grid=(_PREP_GRID,),
        in_specs=[
            pl.BlockSpec((_PREP_BLK, _D), lambda i: (i, 0)),
            pl.BlockSpec((_PREP_BLK, 2), lambda i: (i, 0)),
            pl.BlockSpec((_PREP_BLK, _D), lambda i: (i, 0)),
            pl.BlockSpec((_PREP_BLK, _D), lambda i: (i + 1, 0)),
            pl.BlockSpec((384, _D), lambda i: (0, 0)),
            pl.BlockSpec((1, 384), lambda i: (0, 0)),
        ],
        out_specs=[
            pl.BlockSpec((_PREP_BLK // 8, 2, 8, 128), lambda i: (i, 0, 0, 0)),
            pl.BlockSpec((_PREP_BLK // 8, 4, 8, 128), lambda i: (i, 0, 0, 0)),
            pl.BlockSpec((_PREP_BLK, 256), lambda i: (i, 0)),
        ],
        out_shape=[
            jax.ShapeDtypeStruct((_TQ // 8, 2, 8, 128), jnp.int32),
            jax.ShapeDtypeStruct((_TQ // 8, 4, 8, 128), jnp.float32),
            jax.ShapeDtypeStruct((_TQ, 256), jnp.uint32),
        ],
    )(q2d, rp2d, f2d, f2d, wcat, bcat)


def _sc_body(val_hbm, idx_hbm, w_hbm, out_hbm, idx_v, w_v, g_v, out_v,
             isem, gsem, osem):
    wid = lax.axis_index("s") * 2 + lax.axis_index("c")
    rows_w = jnp.where(wid < 16, 344, 336)
    r0 = jnp.where(wid < 16, wid * 344, 5504 + (wid - 16) * 336)
    s0 = r0 // 8
    nslab = rows_w // 8

    def load_slab(s, slot, sync):
        copy = pltpu.sync_copy if sync else (
            lambda a, b: pltpu.async_copy(a, b, isem))
        copy(idx_hbm.at[s], idx_v.at[slot])
        copy(w_hbm.at[s], w_v.at[slot])

    def fire_gathers(islot, rem, gslot):
        for yp in range(2):
            pltpu.async_copy(
                val_hbm.at[idx_v.at[islot, yp, rem]],
                g_v.at[gslot, pl.ds(yp * 128, 128)],
                gsem,
            )

    def drain_gathers(islot, rem, gslot):
        for yp in range(2):
            pltpu.make_async_copy(
                val_hbm.at[idx_v.at[islot, yp, rem]],
                g_v.at[gslot, pl.ds(yp * 128, 128)],
                gsem,
            ).wait()

    def compute(r, islot, rem, gslot, oslot):
        for h in range(8):
            accs = []
            for c in range(4):
                yp, sl = c // 2, c % 2
                wv = w_v[islot, c, rem, pl.ds(h * 16, 16)]
                a0 = jnp.zeros((16,), jnp.float32)
                a1 = jnp.zeros((16,), jnp.float32)
                for k in range(16):
                    wj = wv[k]
                    g32 = g_v[gslot, yp * 128 + h * 16 + k,
                              pl.ds(sl * 16, 16)]
                    ge = lax.bitcast_convert_type(g32 << 16, jnp.float32)
                    go = lax.bitcast_convert_type(
                        g32 & jnp.uint32(0xFFFF0000), jnp.float32)
                    a0 = a0 + wj * ge
                    a1 = a1 + wj * go
                accs.append((a0, a1))
            out_v[oslot, h, pl.ds(0, 16)] = (
                (accs[0][0] + accs[1][0]) + (accs[2][0] + accs[3][0]))
            out_v[oslot, h, pl.ds(16, 16)] = (
                (accs[0][1] + accs[1][1]) + (accs[2][1] + accs[3][1]))
        pltpu.async_copy(out_v.at[oslot], out_hbm.at[r], osem)

    load_slab(s0, 0, True)
    load_slab(s0 + 1, 1, True)
    for p in range(3):
        fire_gathers(0, p, p)

    def step(i, carry):
        r = r0 + i
        f = i + 3
        islot = lax.rem(i // 8, 2)
        rem = lax.rem(i, 8)
        fslot = lax.rem(f // 8, 2)
        frem = lax.rem(f, 8)
        gslot = lax.rem(i, 4)
        fgslot = lax.rem(f, 4)
        oslot = lax.rem(i, 2)

        drain_gathers(islot, rem, gslot)

        @pl.when((rem == 0) & (i >= 8) & (i // 8 + 1 < nslab))
        def _():
            load_slab(s0 + i // 8 + 1, 1 - islot, False)

        @pl.when((f < rows_w) & (frem == 0) & (f >= 16))
        def _():
            pltpu.make_async_copy(
                idx_hbm.at[s0 + f // 8], idx_v.at[fslot], isem).wait()
            pltpu.make_async_copy(
                w_hbm.at[s0 + f // 8], w_v.at[fslot], isem).wait()

        @pl.when(f < rows_w)
        def _():
            fire_gathers(fslot, frem, fgslot)

        @pl.when(i >= 2)
        def _():
            pltpu.make_async_copy(
                out_v.at[oslot], out_hbm.at[r - 2], osem).wait()

        compute(r, islot, rem, gslot, oslot)
        return carry

    lax.fori_loop(0, rows_w, step, 0)

    pltpu.make_async_copy(
        out_v.at[0], out_hbm.at[r0 + rows_w - 2], osem).wait()
    pltpu.make_async_copy(
        out_v.at[1], out_hbm.at[r0 + rows_w - 1], osem).wait()


@functools.cache
def _sc_gather_fn():
    return pl.kernel(
        _sc_body,
        out_type=jax.ShapeDtypeStruct((_TQ, _H, _HD), jnp.float32),
        mesh=plsc.VectorSubcoreMesh(core_axis_name="c", subcore_axis_name="s"),
        scratch_types=[
            pltpu.VMEM((2, 2, 8, 128), jnp.int32),
            pltpu.VMEM((2, 4, 8, 128), jnp.float32),
            pltpu.VMEM((4, 256, _HD), jnp.uint32),
            pltpu.VMEM((2, _H, _HD), jnp.float32),
            pltpu.SemaphoreType.DMA,
            pltpu.SemaphoreType.DMA,
            pltpu.SemaphoreType.DMA,
        ],
        compiler_params=pltpu.CompilerParams(use_tc_tiling_on_sc=False),
    )


def _sc_gather(val, idx, wgt):
    return _sc_gather_fn()(val, idx, wgt)


def _outproj_body(x_ref, w_ref, b_ref, o_ref):
    o_ref[...] = jnp.dot(x_ref[...], w_ref[...].T,
                         preferred_element_type=jnp.float32) + b_ref[...]


def _outproj(x2d, w_out, b_out2d):
    return pl.pallas_call(
        _outproj_body,
        grid=(_OUT_GRID,),
        in_specs=[
            pl.BlockSpec((_OUT_BLK, _D), lambda i: (i, 0)),
            pl.BlockSpec((_D, _D), lambda i: (0, 0)),
            pl.BlockSpec((1, _D), lambda i: (0, 0)),
        ],
        out_specs=pl.BlockSpec((_OUT_BLK, _D), lambda i: (i, 0)),
        out_shape=jax.ShapeDtypeStruct((_TQ, _D), jnp.float32),
    )(x2d, w_out, b_out2d)


def kernel(query, reference_points, input_flatten, spatial_shapes,
           level_start_index, W_off, b_off, W_att, b_att, W_out, b_out):
    q2d = query.reshape(_TQ, _D)
    rp2d = reference_points.reshape(_TQ, 2).astype(jnp.float32)

    wcat = jnp.concatenate([W_off[0::2], W_off[1::2], W_att], axis=0)
    bcat = jnp.concatenate([b_off[0::2], b_off[1::2], b_att])[None, :]

    f2d = input_flatten.reshape(_TQ, _D)
    idx, wgt, val2d = _prep(q2d, rp2d, f2d, wcat, bcat)

    val = val2d.reshape(_TQ * _H, _HD)
    sampled = _sc_gather(val, idx, wgt)

    out = _outproj(sampled.reshape(_TQ, _D), W_out, b_out[None, :])
    return out.reshape(_NB, _LQ, _D)


# speedup vs baseline: 1.2667x; 1.0431x over previous
import functools

import jax
import jax.numpy as jnp
from jax import lax
from jax.experimental import pallas as pl
from jax.experimental.pallas import tpu as pltpu
from jax.experimental.pallas import tpu_sc as plsc

_D = 256
_H = 8
_L = 4
_P = 4
_HD = 32
_NB = 2
_LQ = 5440
_TQ = _NB * _LQ
_NW = 32
_RPW = _TQ // _NW

_PREP_BLK = 320
_PREP_GRID = _TQ // _PREP_BLK
_BLKS_PER_B = _LQ // _PREP_BLK

_OUT_BLK = 640
_OUT_GRID = _TQ // _OUT_BLK


def _prep_body(q_ref, rp_ref, f_ref, fn_ref, w_ref, b_ref,
               idx_ref, wgt_ref, val_ref):
    batch = pl.program_id(0) // _BLKS_PER_B

    x = f_ref[...]
    xshift = jnp.concatenate([x[1:], fn_ref[pl.ds(0, 1), :]], axis=0)
    xcat = jnp.concatenate([x, xshift], axis=1)
    ci = lax.broadcasted_iota(jnp.int32, (512, 256), 0)
    cj = lax.broadcasted_iota(jnp.int32, (512, 256), 1)
    locol = ((cj % 32) // 16) * 256 + (cj // 32) * 32 + (cj % 16)
    pmat_lo = jnp.where(ci == locol, 1.0, 0.0).astype(jnp.float32)
    pmat_hi = jnp.where(ci == locol + 16, 1.0, 0.0).astype(jnp.float32)
    lo = jnp.dot(xcat, pmat_lo, preferred_element_type=jnp.float32)
    hi = jnp.dot(xcat, pmat_hi, preferred_element_type=jnp.float32)
    lo16 = lax.bitcast_convert_type(lo.astype(jnp.bfloat16), jnp.uint16)
    hi16 = lax.bitcast_convert_type(hi.astype(jnp.bfloat16), jnp.uint16)
    packed = lo16.astype(jnp.uint32) | (hi16.astype(jnp.uint32) << 16)
    for t in range(2):
        val_ref[:, t] = packed[:, t * 128:(t + 1) * 128].reshape(
            _PREP_BLK // 8, 8, 128)

    proj = jnp.dot(q_ref[...], w_ref[...].T, preferred_element_type=jnp.float32)
    proj = proj + b_ref[...]
    offx = proj[:, 0:128]
    offy = proj[:, 128:256]
    logits = proj[:, 256:384]

    logits = logits - jnp.max(logits, axis=1, keepdims=True)
    e = jnp.exp(logits)
    ii = lax.broadcasted_iota(jnp.int32, (128, 128), 0)
    jj = lax.broadcasted_iota(jnp.int32, (128, 128), 1)
    grp = jnp.where((ii // 16) == (jj // 16), 1.0, 0.0).astype(jnp.float32)
    gsum = jnp.dot(e, grp, preferred_element_type=jnp.float32)
    attn = e / gsum

    col = lax.broadcasted_iota(jnp.int32, (1, 128), 1)
    lvl = (col >> 2) & 3
    head = col >> 4
    wl = 64 >> lvl
    hl = 64 >> lvl
    start = jnp.where(lvl == 0, 0,
            jnp.where(lvl == 1, 4096,
            jnp.where(lvl == 2, 5120, 5376)))
    wl_f = wl.astype(jnp.float32)
    hl_f = hl.astype(jnp.float32)

    ix = rp_ref[:, 0:1] * wl_f + offx - 0.5
    iy = rp_ref[:, 1:2] * hl_f + offy - 0.5
    x0f = jnp.floor(ix)
    y0f = jnp.floor(iy)
    wx1 = ix - x0f
    wx0 = 1.0 - wx1
    wy1 = iy - y0f
    wy0 = 1.0 - wy1
    x0 = x0f.astype(jnp.int32)
    y0 = y0f.astype(jnp.int32)

    one = jnp.float32(1.0)
    zero = jnp.float32(0.0)
    xp = jnp.clip(x0, 0, wl - 2)
    x1 = x0 + 1
    s0 = (wx0 * jnp.where(x0 == xp, one, zero)
          + wx1 * jnp.where(x1 == xp, one, zero))
    s1 = (wx0 * jnp.where(x0 == xp + 1, one, zero)
          + wx1 * jnp.where(x1 == xp + 1, one, zero))
    for yp, wy in ((0, wy0), (1, wy1)):
        yv = y0 + yp
        yvalid = jnp.where((yv >= 0) & (yv < hl), one, zero)
        yc = jnp.clip(yv, 0, hl - 1)
        spatial = start + yc * wl + xp
        q = batch * _LQ + spatial
        row = (q >> 3) * 64 + (head >> 2) * 32 + (q & 7) * 4 + (head & 3)
        idx_ref[:, yp] = row.reshape(_PREP_BLK // 8, 8, 128)
        wgt_ref[:, yp * 2] = (attn * s0 * wy * yvalid).reshape(
            _PREP_BLK // 8, 8, 128)
        wgt_ref[:, yp * 2 + 1] = (attn * s1 * wy * yvalid).reshape(
            _PREP_BLK // 8, 8, 128)


def _prep(q2d, rp2d, f2d, wcat, bcat):
    return pl.pallas_call(
        _prep_body,
        grid=(_PREP_GRID,),
        in_specs=[
            pl.BlockSpec((_PREP_BLK, _D), lambda i: (i, 0)),
            pl.BlockSpec((_PREP_BLK, 2), lambda i: (i, 0)),
            pl.BlockSpec((_PREP_BLK, _D), lambda i: (i, 0)),
            pl.BlockSpec((_PREP_BLK, _D), lambda i: (i + 1, 0)),
            pl.BlockSpec((384, _D), lambda i: (0, 0)),
            pl.BlockSpec((1, 384), lambda i: (0, 0)),
        ],
        out_specs=[
            pl.BlockSpec((_PREP_BLK // 8, 2, 8, 128), lambda i: (i, 0, 0, 0)),
            pl.BlockSpec((_PREP_BLK // 8, 4, 8, 128), lambda i: (i, 0, 0, 0)),
            pl.BlockSpec((_PREP_BLK // 8, 2, 8, 128), lambda i: (i, 0, 0, 0)),
        ],
        out_shape=[
            jax.ShapeDtypeStruct((_TQ // 8, 2, 8, 128), jnp.int32),
            jax.ShapeDtypeStruct((_TQ // 8, 4, 8, 128), jnp.float32),
            jax.ShapeDtypeStruct((_TQ // 8, 2, 8, 128), jnp.uint32),
        ],
    )(q2d, rp2d, f2d, f2d, wcat, bcat)


def _sc_body(val_hbm, idx_hbm, w_hbm, out_hbm, idx_v, w_v, g_v, out_v,
             isem, gsem, osem):
    wid = lax.axis_index("s") * 2 + lax.axis_index("c")
    rows_w = jnp.where(wid < 16, 344, 336)
    r0 = jnp.where(wid < 16, wid * 344, 5504 + (wid - 16) * 336)
    s0 = r0 // 8
    nslab = rows_w // 8

    def load_slab(s, slot, sync):
        copy = pltpu.sync_copy if sync else (
            lambda a, b: pltpu.async_copy(a, b, isem))
        copy(idx_hbm.at[s], idx_v.at[slot])
        copy(w_hbm.at[s], w_v.at[slot])

    def fire_gathers(islot, rem, gslot):
        for yp in range(2):
            pltpu.async_copy(
                val_hbm.at[idx_v.at[islot, yp, rem]],
                g_v.at[gslot, pl.ds(yp * 128, 128)],
                gsem,
            )

    def drain_gathers(islot, rem, gslot):
        for yp in range(2):
            pltpu.make_async_copy(
                val_hbm.at[idx_v.at[islot, yp, rem]],
                g_v.at[gslot, pl.ds(yp * 128, 128)],
                gsem,
            ).wait()

    def compute(r, islot, rem, gslot, oslot):
        for h in range(8):
            accs = []
            for c in range(4):
                yp, sl = c // 2, c % 2
                wv = w_v[islot, c, rem, pl.ds(h * 16, 16)]
                a0 = jnp.zeros((16,), jnp.float32)
                a1 = jnp.zeros((16,), jnp.float32)
                for k in range(16):
                    wj = wv[k]
                    g32 = g_v[gslot, yp * 128 + h * 16 + k,
                              pl.ds(sl * 16, 16)]
                    ge = lax.bitcast_convert_type(g32 << 16, jnp.float32)
                    go = lax.bitcast_convert_type(
                        g32 & jnp.uint32(0xFFFF0000), jnp.float32)
                    a0 = a0 + wj * ge
                    a1 = a1 + wj * go
                accs.append((a0, a1))
            out_v[oslot, h, pl.ds(0, 16)] = (
                (accs[0][0] + accs[1][0]) + (accs[2][0] + accs[3][0]))
            out_v[oslot, h, pl.ds(16, 16)] = (
                (accs[0][1] + accs[1][1]) + (accs[2][1] + accs[3][1]))
        pltpu.async_copy(out_v.at[oslot], out_hbm.at[r], osem)

    load_slab(s0, 0, True)
    load_slab(s0 + 1, 1, True)
    for p in range(3):
        fire_gathers(0, p, p)

    def step(i, carry):
        r = r0 + i
        f = i + 3
        islot = lax.rem(i // 8, 2)
        rem = lax.rem(i, 8)
        fslot = lax.rem(f // 8, 2)
        frem = lax.rem(f, 8)
        gslot = lax.rem(i, 4)
        fgslot = lax.rem(f, 4)
        oslot = lax.rem(i, 2)

        drain_gathers(islot, rem, gslot)

        @pl.when((rem == 0) & (i >= 8) & (i // 8 + 1 < nslab))
        def _():
            load_slab(s0 + i // 8 + 1, 1 - islot, False)

        @pl.when((f < rows_w) & (frem == 0) & (f >= 16))
        def _():
            pltpu.make_async_copy(
                idx_hbm.at[s0 + f // 8], idx_v.at[fslot], isem).wait()
            pltpu.make_async_copy(
                w_hbm.at[s0 + f // 8], w_v.at[fslot], isem).wait()

        @pl.when(f < rows_w)
        def _():
            fire_gathers(fslot, frem, fgslot)

        @pl.when(i >= 2)
        def _():
            pltpu.make_async_copy(
                out_v.at[oslot], out_hbm.at[r - 2], osem).wait()

        compute(r, islot, rem, gslot, oslot)
        return carry

    lax.fori_loop(0, rows_w, step, 0)

    pltpu.make_async_copy(
        out_v.at[0], out_hbm.at[r0 + rows_w - 2], osem).wait()
    pltpu.make_async_copy(
        out_v.at[1], out_hbm.at[r0 + rows_w - 1], osem).wait()


@functools.cache
def _sc_gather_fn():
    return pl.kernel(
        _sc_body,
        out_type=jax.ShapeDtypeStruct((_TQ, _H, _HD), jnp.float32),
        mesh=plsc.VectorSubcoreMesh(core_axis_name="c", subcore_axis_name="s"),
        scratch_types=[
            pltpu.VMEM((2, 2, 8, 128), jnp.int32),
            pltpu.VMEM((2, 4, 8, 128), jnp.float32),
            pltpu.VMEM((4, 256, _HD), jnp.uint32),
            pltpu.VMEM((2, _H, _HD), jnp.float32),
            pltpu.SemaphoreType.DMA,
            pltpu.SemaphoreType.DMA,
            pltpu.SemaphoreType.DMA,
        ],
        compiler_params=pltpu.CompilerParams(use_tc_tiling_on_sc=False),
    )


def _sc_gather(val, idx, wgt):
    return _sc_gather_fn()(val, idx, wgt)


def _outproj_body(x_ref, w_ref, b_ref, o_ref):
    o_ref[...] = jnp.dot(x_ref[...], w_ref[...].T,
                         preferred_element_type=jnp.float32) + b_ref[...]


def _outproj(x2d, w_out, b_out2d):
    return pl.pallas_call(
        _outproj_body,
        grid=(_OUT_GRID,),
        in_specs=[
            pl.BlockSpec((_OUT_BLK, _D), lambda i: (i, 0)),
            pl.BlockSpec((_D, _D), lambda i: (0, 0)),
            pl.BlockSpec((1, _D), lambda i: (0, 0)),
        ],
        out_specs=pl.BlockSpec((_OUT_BLK, _D), lambda i: (i, 0)),
        out_shape=jax.ShapeDtypeStruct((_TQ, _D), jnp.float32),
    )(x2d, w_out, b_out2d)


def kernel(query, reference_points, input_flatten, spatial_shapes,
           level_start_index, W_off, b_off, W_att, b_att, W_out, b_out):
    q2d = query.reshape(_TQ, _D)
    rp2d = reference_points.reshape(_TQ, 2).astype(jnp.float32)

    wcat = jnp.concatenate([W_off[0::2], W_off[1::2], W_att], axis=0)
    bcat = jnp.concatenate([b_off[0::2], b_off[1::2], b_att])[None, :]

    f2d = input_flatten.reshape(_TQ, _D)
    idx, wgt, val2d = _prep(q2d, rp2d, f2d, wcat, bcat)

    val = val2d.reshape(_TQ * _H, _HD)
    sampled = _sc_gather(val, idx, wgt)

    out = _outproj(sampled.reshape(_TQ, _D), W_out, b_out[None, :])
    return out.reshape(_NB, _LQ, _D)
